# baseline (speedup 1.0000x reference)
import functools

import jax
import jax.numpy as jnp
from jax.experimental import pallas as pl
from jax.experimental.pallas import tpu as pltpu

N = 10000
D_IN = 128
D_OUT = 128
BM = 400
BS = 2000


def _support_kernel(x_ref, w_ref, s_ref):
    s_ref[...] = jnp.dot(x_ref[...], w_ref[...], preferred_element_type=jnp.float32)


def _agg_kernel(s_ref, b_ref, adj_ref, o_ref):
    acc = jnp.dot(adj_ref[...], s_ref[...], preferred_element_type=jnp.float32)
    o_ref[...] = jnp.maximum(acc + b_ref[...], 0.0)


@jax.jit
def kernel(x, adj, W, b):
    support = pl.pallas_call(
        _support_kernel,
        grid=(N // BS,),
        in_specs=[
            pl.BlockSpec((BS, D_IN), lambda i: (i, 0)),
            pl.BlockSpec((D_IN, D_OUT), lambda i: (0, 0)),
        ],
        out_specs=pl.BlockSpec((BS, D_OUT), lambda i: (i, 0)),
        out_shape=jax.ShapeDtypeStruct((N, D_OUT), jnp.float32),
    )(x, W)
    b2 = b.reshape(1, D_OUT)
    return pl.pallas_call(
        _agg_kernel,
        grid=(N // BM,),
        in_specs=[
            pl.BlockSpec((N, D_OUT), lambda i: (0, 0)),
            pl.BlockSpec((1, D_OUT), lambda i: (0, 0)),
            pl.BlockSpec((BM, N), lambda i: (i, 0)),
        ],
        out_specs=pl.BlockSpec((BM, D_OUT), lambda i: (i, 0)),
        out_shape=jax.ShapeDtypeStruct((N, D_OUT), jnp.float32),
        compiler_params=pltpu.CompilerParams(
            dimension_semantics=("parallel",),
            vmem_limit_bytes=128 * 1024 * 1024,
        ),
    )(support, b2, adj)


# speedup vs baseline: 1.0418x; 1.0418x over previous
import functools

import jax
import jax.numpy as jnp
from jax.experimental import pallas as pl
from jax.experimental.pallas import tpu as pltpu

N = 10000
H = N // 2
D_IN = 128
D_OUT = 128
BM = 200


def _gcn_kernel(x_ref, w_ref, b_ref, a0_ref, a1_ref, o_ref, s_ref):
    @pl.when(pl.program_id(0) == 0)
    def _():
        s_ref[...] = jnp.dot(
            x_ref[...], w_ref[...], preferred_element_type=jnp.float32
        )

    s = s_ref[...]
    acc0 = jnp.dot(a0_ref[0], s, preferred_element_type=jnp.float32)
    acc1 = jnp.dot(a1_ref[0], s, preferred_element_type=jnp.float32)
    o_ref[0] = jnp.maximum(acc0 + b_ref[...], 0.0)
    o_ref[1] = jnp.maximum(acc1 + b_ref[...], 0.0)


@jax.jit
def kernel(x, adj, W, b):
    b2 = b.reshape(1, D_OUT)
    adj_r = adj.reshape(2, H, N)
    out = pl.pallas_call(
        _gcn_kernel,
        grid=(H // BM,),
        in_specs=[
            pl.BlockSpec((N, D_IN), lambda i: (0, 0)),
            pl.BlockSpec((D_IN, D_OUT), lambda i: (0, 0)),
            pl.BlockSpec((1, D_OUT), lambda i: (0, 0)),
            pl.BlockSpec((1, BM, N), lambda i: (0, i, 0)),
            pl.BlockSpec((1, BM, N), lambda i: (1, i, 0)),
        ],
        out_specs=pl.BlockSpec((2, BM, D_OUT), lambda i: (0, i, 0)),
        out_shape=jax.ShapeDtypeStruct((2, H, D_OUT), jnp.float32),
        scratch_shapes=[pltpu.VMEM((N, D_OUT), jnp.float32)],
        compiler_params=pltpu.CompilerParams(
            vmem_limit_bytes=128 * 1024 * 1024,
        ),
    )(x, W, b2, adj_r, adj_r)
    return out.reshape(N, D_OUT)


# speedup vs baseline: 1.0564x; 1.0140x over previous
import functools

import jax
import jax.numpy as jnp
from jax.experimental import pallas as pl
from jax.experimental.pallas import tpu as pltpu

N = 10000
D_IN = 128
D_OUT = 128
BM = 400


def _gcn_kernel(x_ref, w_ref, b_ref, adj_ref, o_ref, s_ref):
    @pl.when(pl.program_id(0) == 0)
    def _():
        s_ref[...] = jnp.dot(
            x_ref[...], w_ref[...], preferred_element_type=jnp.float32
        )

    acc = jnp.dot(adj_ref[...], s_ref[...], preferred_element_type=jnp.float32)
    o_ref[...] = jnp.maximum(acc + b_ref[...], 0.0)


@jax.jit
def kernel(x, adj, W, b):
    b2 = b.reshape(1, D_OUT)
    return pl.pallas_call(
        _gcn_kernel,
        grid=(N // BM,),
        in_specs=[
            pl.BlockSpec((N, D_IN), lambda i: (0, 0)),
            pl.BlockSpec((D_IN, D_OUT), lambda i: (0, 0)),
            pl.BlockSpec((1, D_OUT), lambda i: (0, 0)),
            pl.BlockSpec((BM, N), lambda i: (i, 0)),
        ],
        out_specs=pl.BlockSpec((BM, D_OUT), lambda i: (i, 0)),
        out_shape=jax.ShapeDtypeStruct((N, D_OUT), jnp.float32),
        scratch_shapes=[pltpu.VMEM((N, D_OUT), jnp.float32)],
        compiler_params=pltpu.CompilerParams(
            vmem_limit_bytes=128 * 1024 * 1024,
        ),
    )(x, W, b2, adj)
